# Initial kernel scaffold; baseline (speedup 1.0000x reference)
import jax
import jax.numpy as jnp
from jax.experimental import pallas as pl


def kernel(x, edge_index, batch, params):
    raise NotImplementedError("write your pallas kernel here")


# speedup vs baseline: 5.0005x; 5.0005x over previous
import functools

import numpy as np
import jax
import jax.numpy as jnp
from jax import lax
from jax.experimental import pallas as pl
from jax.experimental.pallas import tpu as pltpu
from jax.experimental.pallas import tpu_sc as plsc

N = 10000
E = 320000
HID = 128
NROWS = 10240
TROWS = NROWS // 16
CHUNK = 64
NCH = 160
NTILES = 32
EPAD = NTILES * NCH * CHUNK
RB = 400
GRID = N // RB


def _sc_agg_impl(round0, h, sdr, parts, acc, sd_b, msk_b, rows_v, zrow_v,
                 isems, gsems, ssems):
    c = lax.axis_index("c")
    s = lax.axis_index("s")
    base = (c * 16 + s) * NCH

    def zbody(i, carry):
        for l in range(8):
            zrow_v[i, pl.ds(l * 16, 16)] = jnp.zeros((16,), jnp.float32)
        return carry
    lax.fori_loop(0, 16, zbody, 0)

    rb = s * TROWS

    def istart(j, q):
        pltpu.async_copy(sdr.at[base + j], sd_b.at[q], isems[q])

    def iwait(q):
        pltpu.make_async_copy(sdr.at[0], sd_b.at[q], isems[q]).wait()

    def mcomp(j, q):
        for l in range(CHUNK // 16):
            s16 = sd_b[q, 0, pl.ds(l * 16, 16)]
            d16 = sd_b[q, 1, pl.ds(l * 16, 16)]
            t16 = jnp.int32(N) + jnp.bitwise_and(d16, jnp.int32(127))
            msk_b[q, pl.ds(l * 16, 16)] = jnp.where(s16 == d16, t16, d16)

    def gstart(j, b, q):
        pltpu.async_copy(h.at[sd_b.at[q, 0]], rows_v.at[b], gsems[b])

    def gwait(b):
        pltpu.make_async_copy(h.at[pl.ds(0, CHUNK)], rows_v.at[b],
                              gsems[b]).wait()

    def sstart(j, b, q):
        idx = msk_b.at[q] if round0 else sd_b.at[q, 1]
        pltpu.async_copy(rows_v.at[b], acc.at[idx], ssems[b], add=True)

    def swait(b):
        pltpu.make_async_copy(rows_v.at[b], acc.at[pl.ds(0, CHUNK)],
                              ssems[b]).wait()

    istart(0, 0)
    istart(1, 1)
    istart(2, 2)

    def ibody(k, carry):
        pltpu.sync_copy(zrow_v, acc.at[pl.ds(rb + k * 16, 16)])
        return carry
    lax.fori_loop(0, TROWS // 16, ibody, 0)

    iwait(0)
    if round0:
        mcomp(0, 0)
    gstart(0, 0, 0)
    plsc.subcore_barrier()

    def lbody(g, carry):
        for b4 in range(4):
            j = 4 * g + b4
            b = b4 % 2
            q = b4
            qn = (b4 + 1) % 4
            nz = g >= 1 if b4 == 0 else True

            gwait(b)
            sstart(j, b, q)
            if nz is True:
                swait(1 - b)
            else:
                pl.when(nz)(lambda: swait(1 - b))

            def nxt():
                iwait(qn)
                if round0:
                    mcomp(j + 1, qn)
                gstart(j + 1, 1 - b, qn)
            if b4 < 3:
                nxt()
            else:
                pl.when(g <= NCH // 4 - 2)(nxt)

            def pre():
                istart(j + 3, (b4 + 3) % 4)
            if b4 == 0:
                pre()
            else:
                pl.when(g <= NCH // 4 - 2)(pre)
        return carry
    lax.fori_loop(0, NCH // 4, lbody, 0)

    swait((NCH - 1) % 2)
    plsc.subcore_barrier()

    pltpu.sync_copy(acc.at[pl.ds(rb, TROWS)], parts.at[c, pl.ds(rb, TROWS)])


def _make_sc_agg(round0):
    mesh = plsc.VectorSubcoreMesh(core_axis_name="c", subcore_axis_name="s",
                                  num_cores=2, num_subcores=16)
    out_type = [jax.ShapeDtypeStruct((2, NROWS, HID), jnp.float32)]
    scratch = [
        pltpu.VMEM_SHARED((NROWS, HID), jnp.float32),
        pltpu.VMEM((4, 2, CHUNK), jnp.int32),
        pltpu.VMEM((2, CHUNK, HID), jnp.float32),
        pltpu.VMEM((16, HID), jnp.float32),
        [pltpu.SemaphoreType.DMA] * 4,
        [pltpu.SemaphoreType.DMA] * 2,
        [pltpu.SemaphoreType.DMA] * 2,
    ]
    if round0:
        scratch.append(pltpu.VMEM((4, CHUNK), jnp.int32))

        def body(h, sdr, parts, acc, sd_b, rows_v, zrow_v, isems,
                 gsems, ssems, msk_b):
            _sc_agg_impl(True, h, sdr, parts, acc, sd_b, msk_b, rows_v,
                         zrow_v, isems, gsems, ssems)
    else:
        def body(h, sdr, parts, acc, sd_b, rows_v, zrow_v, isems, gsems,
                 ssems):
            _sc_agg_impl(False, h, sdr, parts, acc, sd_b, None, rows_v,
                         zrow_v, isems, gsems, ssems)

    return pl.kernel(body, out_type=out_type, mesh=mesh,
                     scratch_types=scratch)


def _a_body(x_ref, m_ref, br_ref, wls_ref, bls_ref, wln_ref, bln_ref,
            sx_ref, xn_ref):
    h0 = jnp.dot(x_ref[...], m_ref[...],
                 preferred_element_type=jnp.float32) + br_ref[...]
    h0 = jnp.maximum(h0, 0.0)
    sx_ref[...] = jnp.dot(h0, wls_ref[...],
                          preferred_element_type=jnp.float32) + bls_ref[...]
    xn_ref[...] = jnp.dot(h0, wln_ref[...],
                          preferred_element_type=jnp.float32) + bln_ref[...]


def _b0_body(sx_ref, p0_ref, p1_ref, h_ref):
    h_ref[...] = sx_ref[...] + p0_ref[0] + p1_ref[0]


def _sage_mid_body(p0, p1, c0, c1, h_ref, wl_ref, bl_ref, wr_ref, g_ref,
                   b_ref, out_ref):
    sagg = p0[0] + p1[0]
    cnt = c0[0][:, 0:1] + c1[0][:, 0:1]
    mean = sagg / jnp.maximum(cnt, 1.0)
    t = (jnp.dot(mean, wl_ref[...], preferred_element_type=jnp.float32)
         + bl_ref[...]
         + jnp.dot(h_ref[...], wr_ref[...],
                   preferred_element_type=jnp.float32))
    hr = jnp.maximum(t, 0.0)
    mu = jnp.mean(hr, axis=-1, keepdims=True)
    var = jnp.mean((hr - mu) ** 2, axis=-1, keepdims=True)
    out_ref[...] = (hr - mu) / jnp.sqrt(var + 1e-5) * g_ref[...] + b_ref[...]


def _sage_last_body(p0, p1, c0, c1, h_ref, wl_ref, bl_ref, wr_ref, w1_ref,
                    b1_ref, w2_ref, b2_ref, emb_ref, lp_ref):
    sagg = p0[0] + p1[0]
    cnt = c0[0][:, 0:1] + c1[0][:, 0:1]
    mean = sagg / jnp.maximum(cnt, 1.0)
    t = (jnp.dot(mean, wl_ref[...], preferred_element_type=jnp.float32)
         + bl_ref[...]
         + jnp.dot(h_ref[...], wr_ref[...],
                   preferred_element_type=jnp.float32))
    emb_ref[...] = t
    hr = jnp.maximum(t, 0.0)
    z1 = jnp.dot(hr, w1_ref[...],
                 preferred_element_type=jnp.float32) + b1_ref[...]
    z = jnp.dot(z1, w2_ref[...],
                preferred_element_type=jnp.float32) + b2_ref[...]
    m = jnp.max(z, axis=-1, keepdims=True)
    lse = m + jnp.log(jnp.sum(jnp.exp(z - m), axis=-1, keepdims=True))
    lp_ref[...] = z - lse


def _rows_spec(width):
    return pl.BlockSpec((RB, width), lambda i: (i, 0))


def _full_spec(shape):
    nd = len(shape)
    return pl.BlockSpec(shape, lambda i: (0,) * nd)


def _part_spec(part, width):
    return pl.BlockSpec((1, RB, width), lambda i, _p=part: (_p, i, 0))


def _conv_as_matmul(Wcs):
    co, ci, di, dj, oi, oj = np.meshgrid(
        np.arange(4), np.arange(4), np.arange(3), np.arange(3),
        np.arange(8), np.arange(8), indexing="ij")
    rows = (ci * 100 + (oi + di) * 10 + (oj + dj)).ravel()
    cols = (co * 64 + oi * 8 + oj).ravel()
    vals = Wcs[co.ravel(), ci.ravel(), di.ravel(), dj.ravel()]
    return jnp.zeros((400, 256), jnp.float32).at[rows, cols].set(vals)


def kernel(x, edge_index, batch, params):
    f32 = jnp.float32
    x2d = x.reshape(N, 400)
    M = _conv_as_matmul(params["Wcs"])
    br = jnp.repeat(params["bcs"], 64).reshape(1, 256)

    npad = EPAD - E
    pad_src = (jnp.arange(npad, dtype=jnp.int32) * 7919) % N
    pad_dst = N + jnp.arange(npad, dtype=jnp.int32) % (NROWS - N)
    src_p = jnp.concatenate([edge_index[0], pad_src]).reshape(EPAD // CHUNK,
                                                              1, CHUNK)
    dst_p = jnp.concatenate([edge_index[1], pad_dst]).reshape(EPAD // CHUNK,
                                                              1, CHUNK)
    sd_p = jnp.concatenate([src_p, dst_p], axis=1)

    sx, xn = pl.pallas_call(
        _a_body,
        grid=(GRID,),
        in_specs=[
            _rows_spec(400),
            _full_spec((400, 256)), _full_spec((1, 256)),
            _full_spec((256, HID)), _full_spec((1, HID)),
            _full_spec((256, HID)), _full_spec((1, HID)),
        ],
        out_specs=[_rows_spec(HID), _rows_spec(HID)],
        out_shape=[jax.ShapeDtypeStruct((N, HID), f32)] * 2,
    )(x2d, M, br,
      params["W_lin_self"], params["b_lin_self"].reshape(1, HID),
      params["W_lin"], params["b_lin"].reshape(1, HID))

    sc_agg = _make_sc_agg(False)

    (parts0,) = _make_sc_agg(True)(xn, sd_p)
    (cnt,) = sc_agg(jnp.ones((N, HID), f32), sd_p)

    h = pl.pallas_call(
        _b0_body,
        grid=(GRID,),
        in_specs=[_rows_spec(HID), _part_spec(0, HID), _part_spec(1, HID)],
        out_specs=_rows_spec(HID),
        out_shape=jax.ShapeDtypeStruct((N, HID), f32),
    )(sx, parts0, parts0)

    hcur = h
    emb = lp = None
    for i in (1, 2, 3):
        (parts,) = sc_agg(hcur, sd_p)
        common_in = [_part_spec(0, HID), _part_spec(1, HID),
                     _part_spec(0, HID), _part_spec(1, HID), _rows_spec(HID),
                     _full_spec((HID, HID)), _full_spec((1, HID)),
                     _full_spec((HID, HID))]
        wl = params["Wl%d" % i]
        bl = params["bl%d" % i].reshape(1, HID)
        wr = params["Wr%d" % i]
        if i != 3:
            hcur = pl.pallas_call(
                _sage_mid_body,
                grid=(GRID,),
                in_specs=common_in + [_full_spec((1, HID)),
                                      _full_spec((1, HID))],
                out_specs=_rows_spec(HID),
                out_shape=jax.ShapeDtypeStruct((N, HID), f32),
            )(parts, parts, cnt, cnt, hcur, wl, bl, wr,
              params["ln%d_g" % i].reshape(1, HID),
              params["ln%d_b" % i].reshape(1, HID))
        else:
            emb, lp = pl.pallas_call(
                _sage_last_body,
                grid=(GRID,),
                in_specs=common_in + [
                    _full_spec((HID, HID)), _full_spec((1, HID)),
                    _full_spec((HID, 16)), _full_spec((1, 16))],
                out_specs=[_rows_spec(HID), _rows_spec(16)],
                out_shape=[jax.ShapeDtypeStruct((N, HID), f32),
                           jax.ShapeDtypeStruct((N, 16), f32)],
            )(parts, parts, cnt, cnt, hcur, wl, bl, wr,
              params["W1"], params["b1"].reshape(1, HID),
              params["W2"], params["b2"].reshape(1, 16))

    return emb, lp


# speedup vs baseline: 6.5003x; 1.2999x over previous
import functools

import numpy as np
import jax
import jax.numpy as jnp
from jax import lax
from jax.experimental import pallas as pl
from jax.experimental.pallas import tpu as pltpu
from jax.experimental.pallas import tpu_sc as plsc

N = 10000
E = 320000
HID = 128
NROWS = 10240
TROWS = NROWS // 16
CHUNK = 64
NCH = 160
NTILES = 32
EPAD = NTILES * NCH * CHUNK
RB = 400
GRID = N // RB


def _sc_agg_impl(round0, h, sdr, parts, acc, sd_b, msk_b, rows_v, zrow_v,
                 isems, gsems, ssems):
    c = lax.axis_index("c")
    s = lax.axis_index("s")
    base = (c * 16 + s) * NCH

    def zbody(i, carry):
        for l in range(8):
            zrow_v[i, pl.ds(l * 16, 16)] = jnp.zeros((16,), jnp.float32)
        return carry
    lax.fori_loop(0, 16, zbody, 0)

    rb = s * TROWS

    def istart(j, q):
        pltpu.async_copy(sdr.at[base + j], sd_b.at[q], isems[q])

    def iwait(q):
        pltpu.make_async_copy(sdr.at[0], sd_b.at[q], isems[q]).wait()

    def mcomp(q, mb):
        for l in range(CHUNK // 16):
            s16 = sd_b[q, 0, pl.ds(l * 16, 16)]
            d16 = sd_b[q, 1, pl.ds(l * 16, 16)]
            t16 = jnp.int32(N) + jnp.bitwise_and(d16, jnp.int32(127))
            msk_b[mb, pl.ds(l * 16, 16)] = jnp.where(s16 == d16, t16, d16)

    def gstart(j, b, q):
        pltpu.async_copy(h.at[sd_b.at[q, 0]], rows_v.at[b], gsems[b])

    def gwait(b):
        pltpu.make_async_copy(h.at[pl.ds(0, CHUNK)], rows_v.at[b],
                              gsems[b]).wait()

    def sstart(j, b, q):
        idx = msk_b.at[b] if round0 else sd_b.at[q, 1]
        pltpu.async_copy(rows_v.at[b], acc.at[idx], ssems[b], add=True)

    def swait(b):
        pltpu.make_async_copy(rows_v.at[b], acc.at[pl.ds(0, CHUNK)],
                              ssems[b]).wait()

    for q0 in range(6):
        istart(q0, q0)

    def ibody(k, carry):
        pltpu.sync_copy(zrow_v, acc.at[pl.ds(rb + k * 16, 16)])
        return carry
    lax.fori_loop(0, TROWS // 16, ibody, 0)

    for j0 in range(2):
        iwait(j0)
        if round0:
            mcomp(j0, j0)
        gstart(j0, j0, j0)
    plsc.subcore_barrier()

    def lbody(g, carry):
        for b8 in range(8):
            j = 8 * g + b8
            b = b8 % 4
            q = b8

            gwait(b)
            sstart(j, b, q)
            if b8 >= 2:
                swait((b8 - 2) % 4)
            else:
                pl.when(g >= 1)(lambda bb=(b8 - 2) % 4: swait(bb))

            def nxt(jq=(b8 + 2) % 8, jb=(b8 + 2) % 4, jj=j + 2):
                iwait(jq)
                if round0:
                    mcomp(jq, jb)
                gstart(jj, jb, jq)
            if b8 < 6:
                nxt()
            else:
                pl.when(g <= NCH // 8 - 2)(nxt)

            def pre(pq=(b8 + 6) % 8, pj=j + 6):
                istart(pj, pq)
            if b8 < 2:
                pre()
            else:
                pl.when(g <= NCH // 8 - 2)(pre)
        return carry
    lax.fori_loop(0, NCH // 8, lbody, 0)

    swait((NCH - 2) % 4)
    swait((NCH - 1) % 4)
    plsc.subcore_barrier()

    pltpu.sync_copy(acc.at[pl.ds(rb, TROWS)], parts.at[c, pl.ds(rb, TROWS)])


def _make_sc_agg(round0):
    mesh = plsc.VectorSubcoreMesh(core_axis_name="c", subcore_axis_name="s",
                                  num_cores=2, num_subcores=16)
    out_type = [jax.ShapeDtypeStruct((2, NROWS, HID), jnp.float32)]
    scratch = [
        pltpu.VMEM_SHARED((NROWS, HID), jnp.float32),
        pltpu.VMEM((8, 2, CHUNK), jnp.int32),
        pltpu.VMEM((4, CHUNK, HID), jnp.float32),
        pltpu.VMEM((16, HID), jnp.float32),
        [pltpu.SemaphoreType.DMA] * 8,
        [pltpu.SemaphoreType.DMA] * 4,
        [pltpu.SemaphoreType.DMA] * 4,
    ]
    if round0:
        scratch.append(pltpu.VMEM((4, CHUNK), jnp.int32))

        def body(h, sdr, parts, acc, sd_b, rows_v, zrow_v, isems,
                 gsems, ssems, msk_b):
            _sc_agg_impl(True, h, sdr, parts, acc, sd_b, msk_b, rows_v,
                         zrow_v, isems, gsems, ssems)
    else:
        def body(h, sdr, parts, acc, sd_b, rows_v, zrow_v, isems, gsems,
                 ssems):
            _sc_agg_impl(False, h, sdr, parts, acc, sd_b, None, rows_v,
                         zrow_v, isems, gsems, ssems)

    return pl.kernel(body, out_type=out_type, mesh=mesh,
                     scratch_types=scratch)


def _a_body(x_ref, m_ref, br_ref, wls_ref, bls_ref, wln_ref, bln_ref,
            sx_ref, xn_ref):
    h0 = jnp.dot(x_ref[...], m_ref[...],
                 preferred_element_type=jnp.float32) + br_ref[...]
    h0 = jnp.maximum(h0, 0.0)
    sx_ref[...] = jnp.dot(h0, wls_ref[...],
                          preferred_element_type=jnp.float32) + bls_ref[...]
    xn_ref[...] = jnp.dot(h0, wln_ref[...],
                          preferred_element_type=jnp.float32) + bln_ref[...]


def _b0_body(sx_ref, p0_ref, p1_ref, h_ref):
    h_ref[...] = sx_ref[...] + p0_ref[0] + p1_ref[0]


def _sage_mid_body(p0, p1, c0, c1, h_ref, wl_ref, bl_ref, wr_ref, g_ref,
                   b_ref, out_ref):
    sagg = p0[0] + p1[0]
    cnt = c0[0][:, 0:1] + c1[0][:, 0:1]
    mean = sagg / jnp.maximum(cnt, 1.0)
    t = (jnp.dot(mean, wl_ref[...], preferred_element_type=jnp.float32)
         + bl_ref[...]
         + jnp.dot(h_ref[...], wr_ref[...],
                   preferred_element_type=jnp.float32))
    hr = jnp.maximum(t, 0.0)
    mu = jnp.mean(hr, axis=-1, keepdims=True)
    var = jnp.mean((hr - mu) ** 2, axis=-1, keepdims=True)
    out_ref[...] = (hr - mu) / jnp.sqrt(var + 1e-5) * g_ref[...] + b_ref[...]


def _sage_last_body(p0, p1, c0, c1, h_ref, wl_ref, bl_ref, wr_ref, w1_ref,
                    b1_ref, w2_ref, b2_ref, emb_ref, lp_ref):
    sagg = p0[0] + p1[0]
    cnt = c0[0][:, 0:1] + c1[0][:, 0:1]
    mean = sagg / jnp.maximum(cnt, 1.0)
    t = (jnp.dot(mean, wl_ref[...], preferred_element_type=jnp.float32)
         + bl_ref[...]
         + jnp.dot(h_ref[...], wr_ref[...],
                   preferred_element_type=jnp.float32))
    emb_ref[...] = t
    hr = jnp.maximum(t, 0.0)
    z1 = jnp.dot(hr, w1_ref[...],
                 preferred_element_type=jnp.float32) + b1_ref[...]
    z = jnp.dot(z1, w2_ref[...],
                preferred_element_type=jnp.float32) + b2_ref[...]
    m = jnp.max(z, axis=-1, keepdims=True)
    lse = m + jnp.log(jnp.sum(jnp.exp(z - m), axis=-1, keepdims=True))
    lp_ref[...] = z - lse


def _rows_spec(width):
    return pl.BlockSpec((RB, width), lambda i: (i, 0))


def _full_spec(shape):
    nd = len(shape)
    return pl.BlockSpec(shape, lambda i: (0,) * nd)


def _part_spec(part, width):
    return pl.BlockSpec((1, RB, width), lambda i, _p=part: (_p, i, 0))


def _conv_as_matmul(Wcs):
    co, ci, di, dj, oi, oj = np.meshgrid(
        np.arange(4), np.arange(4), np.arange(3), np.arange(3),
        np.arange(8), np.arange(8), indexing="ij")
    rows = (ci * 100 + (oi + di) * 10 + (oj + dj)).ravel()
    cols = (co * 64 + oi * 8 + oj).ravel()
    vals = Wcs[co.ravel(), ci.ravel(), di.ravel(), dj.ravel()]
    return jnp.zeros((400, 256), jnp.float32).at[rows, cols].set(vals)


def kernel(x, edge_index, batch, params):
    f32 = jnp.float32
    x2d = x.reshape(N, 400)
    M = _conv_as_matmul(params["Wcs"])
    br = jnp.repeat(params["bcs"], 64).reshape(1, 256)

    npad = EPAD - E
    pad_src = (jnp.arange(npad, dtype=jnp.int32) * 7919) % N
    pad_dst = N + jnp.arange(npad, dtype=jnp.int32) % (NROWS - N)
    src_p = jnp.concatenate([edge_index[0], pad_src]).reshape(EPAD // CHUNK,
                                                              1, CHUNK)
    dst_p = jnp.concatenate([edge_index[1], pad_dst]).reshape(EPAD // CHUNK,
                                                              1, CHUNK)
    sd_p = jnp.concatenate([src_p, dst_p], axis=1)

    sx, xn = pl.pallas_call(
        _a_body,
        grid=(GRID,),
        in_specs=[
            _rows_spec(400),
            _full_spec((400, 256)), _full_spec((1, 256)),
            _full_spec((256, HID)), _full_spec((1, HID)),
            _full_spec((256, HID)), _full_spec((1, HID)),
        ],
        out_specs=[_rows_spec(HID), _rows_spec(HID)],
        out_shape=[jax.ShapeDtypeStruct((N, HID), f32)] * 2,
    )(x2d, M, br,
      params["W_lin_self"], params["b_lin_self"].reshape(1, HID),
      params["W_lin"], params["b_lin"].reshape(1, HID))

    sc_agg = _make_sc_agg(False)

    (parts0,) = _make_sc_agg(True)(xn, sd_p)
    (cnt,) = sc_agg(jnp.ones((N, HID), f32), sd_p)

    h = pl.pallas_call(
        _b0_body,
        grid=(GRID,),
        in_specs=[_rows_spec(HID), _part_spec(0, HID), _part_spec(1, HID)],
        out_specs=_rows_spec(HID),
        out_shape=jax.ShapeDtypeStruct((N, HID), f32),
    )(sx, parts0, parts0)

    hcur = h
    emb = lp = None
    for i in (1, 2, 3):
        (parts,) = sc_agg(hcur, sd_p)
        common_in = [_part_spec(0, HID), _part_spec(1, HID),
                     _part_spec(0, HID), _part_spec(1, HID), _rows_spec(HID),
                     _full_spec((HID, HID)), _full_spec((1, HID)),
                     _full_spec((HID, HID))]
        wl = params["Wl%d" % i]
        bl = params["bl%d" % i].reshape(1, HID)
        wr = params["Wr%d" % i]
        if i != 3:
            hcur = pl.pallas_call(
                _sage_mid_body,
                grid=(GRID,),
                in_specs=common_in + [_full_spec((1, HID)),
                                      _full_spec((1, HID))],
                out_specs=_rows_spec(HID),
                out_shape=jax.ShapeDtypeStruct((N, HID), f32),
            )(parts, parts, cnt, cnt, hcur, wl, bl, wr,
              params["ln%d_g" % i].reshape(1, HID),
              params["ln%d_b" % i].reshape(1, HID))
        else:
            emb, lp = pl.pallas_call(
                _sage_last_body,
                grid=(GRID,),
                in_specs=common_in + [
                    _full_spec((HID, HID)), _full_spec((1, HID)),
                    _full_spec((HID, 16)), _full_spec((1, 16))],
                out_specs=[_rows_spec(HID), _rows_spec(16)],
                out_shape=[jax.ShapeDtypeStruct((N, HID), f32),
                           jax.ShapeDtypeStruct((N, 16), f32)],
            )(parts, parts, cnt, cnt, hcur, wl, bl, wr,
              params["W1"], params["b1"].reshape(1, HID),
              params["W2"], params["b2"].reshape(1, 16))

    return emb, lp


# speedup vs baseline: 7.0870x; 1.0902x over previous
import functools

import numpy as np
import jax
import jax.numpy as jnp
from jax import lax
from jax.experimental import pallas as pl
from jax.experimental.pallas import tpu as pltpu
from jax.experimental.pallas import tpu_sc as plsc

N = 10000
E = 320000
HID = 128
NROWS = 10240
TROWS = NROWS // 16
CHUNK = 80
NCH = 128
NTILES = 32
EPAD = NTILES * NCH * CHUNK
RB = 400
GRID = N // RB


def _sc_agg_impl(round0, h, sdr, parts, acc, sd_b, msk_b, rows_v, zrow_v,
                 isems, gsems, ssems):
    c = lax.axis_index("c")
    s = lax.axis_index("s")
    base = (c * 16 + s) * NCH

    def zbody(i, carry):
        for l in range(8):
            zrow_v[i, pl.ds(l * 16, 16)] = jnp.zeros((16,), jnp.float32)
        return carry
    lax.fori_loop(0, 16, zbody, 0)

    rb = s * TROWS

    def istart(j, q):
        pltpu.async_copy(sdr.at[base + j], sd_b.at[q], isems[q])

    def iwait(q):
        pltpu.make_async_copy(sdr.at[0], sd_b.at[q], isems[q]).wait()

    def mcomp(q, mb):
        for l in range(CHUNK // 16):
            s16 = sd_b[q, 0, pl.ds(l * 16, 16)]
            d16 = sd_b[q, 1, pl.ds(l * 16, 16)]
            t16 = jnp.int32(N) + jnp.bitwise_and(d16, jnp.int32(127))
            msk_b[mb, pl.ds(l * 16, 16)] = jnp.where(s16 == d16, t16, d16)

    def gstart(j, b, q):
        pltpu.async_copy(h.at[sd_b.at[q, 0]], rows_v.at[b], gsems[b])

    def gwait(b):
        pltpu.make_async_copy(h.at[pl.ds(0, CHUNK)], rows_v.at[b],
                              gsems[b]).wait()

    def sstart(j, b, q):
        idx = msk_b.at[b] if round0 else sd_b.at[q, 1]
        pltpu.async_copy(rows_v.at[b], acc.at[idx], ssems[b], add=True)

    def swait(b):
        pltpu.make_async_copy(rows_v.at[b], acc.at[pl.ds(0, CHUNK)],
                              ssems[b]).wait()

    for q0 in range(6):
        istart(q0, q0)

    def ibody(k, carry):
        pltpu.sync_copy(zrow_v, acc.at[pl.ds(rb + k * 16, 16)])
        return carry
    lax.fori_loop(0, TROWS // 16, ibody, 0)

    for j0 in range(2):
        iwait(j0)
        if round0:
            mcomp(j0, j0)
        gstart(j0, j0, j0)
    plsc.subcore_barrier()

    def lbody(g, carry):
        for b8 in range(8):
            j = 8 * g + b8
            b = b8 % 4
            q = b8

            gwait(b)
            sstart(j, b, q)
            if b8 >= 2:
                swait((b8 - 2) % 4)
            else:
                pl.when(g >= 1)(lambda bb=(b8 - 2) % 4: swait(bb))

            def nxt(jq=(b8 + 2) % 8, jb=(b8 + 2) % 4, jj=j + 2):
                iwait(jq)
                if round0:
                    mcomp(jq, jb)
                gstart(jj, jb, jq)
            if b8 < 6:
                nxt()
            else:
                pl.when(g <= NCH // 8 - 2)(nxt)

            def pre(pq=(b8 + 6) % 8, pj=j + 6):
                istart(pj, pq)
            if b8 < 2:
                pre()
            else:
                pl.when(g <= NCH // 8 - 2)(pre)
        return carry
    lax.fori_loop(0, NCH // 8, lbody, 0)

    swait((NCH - 2) % 4)
    swait((NCH - 1) % 4)
    plsc.subcore_barrier()

    pltpu.sync_copy(acc.at[pl.ds(rb, TROWS)], parts.at[c, pl.ds(rb, TROWS)])


def _sc_cnt_impl(sdr, parts, acc, sd_b, ones_v, zrow_v, isems, ssems):
    c = lax.axis_index("c")
    s = lax.axis_index("s")
    base = (c * 16 + s) * NCH
    rb = s * TROWS

    for q0 in range(4):
        pltpu.async_copy(sdr.at[base + q0], sd_b.at[q0], isems[q0])

    def zbody(i, carry):
        for l in range(8):
            zrow_v[i, pl.ds(l * 16, 16)] = jnp.zeros((16,), jnp.float32)
        return carry
    lax.fori_loop(0, 16, zbody, 0)

    def obody(i, carry):
        for l in range(8):
            ones_v[i, pl.ds(l * 16, 16)] = jnp.ones((16,), jnp.float32)
        return carry
    lax.fori_loop(0, CHUNK, obody, 0)

    def ibody(k, carry):
        pltpu.sync_copy(zrow_v, acc.at[pl.ds(rb + k * 16, 16)])
        return carry
    lax.fori_loop(0, TROWS // 16, ibody, 0)

    plsc.subcore_barrier()

    def swait(s4):
        pltpu.make_async_copy(ones_v, acc.at[pl.ds(0, CHUNK)],
                              ssems[s4]).wait()

    def lbody(g, carry):
        for b8 in range(8):
            j = 8 * g + b8
            q = b8
            s4 = b8 % 4
            if b8 >= 4:
                swait(s4)
            else:
                pl.when(g >= 1)(lambda ss=s4: swait(ss))
            pltpu.make_async_copy(sdr.at[0], sd_b.at[q], isems[q]).wait()
            pltpu.async_copy(ones_v, acc.at[sd_b.at[q, 1]], ssems[s4],
                             add=True)

            def pre(pq=(b8 + 4) % 8, pj=j + 4):
                pltpu.async_copy(sdr.at[base + pj], sd_b.at[pq], isems[pq])
            if b8 < 4:
                pre()
            else:
                pl.when(g <= NCH // 8 - 2)(pre)
        return carry
    lax.fori_loop(0, NCH // 8, lbody, 0)

    for s4 in range(4):
        swait(s4)
    plsc.subcore_barrier()
    pltpu.sync_copy(acc.at[pl.ds(rb, TROWS)], parts.at[c, pl.ds(rb, TROWS)])


def _make_sc_cnt():
    mesh = plsc.VectorSubcoreMesh(core_axis_name="c", subcore_axis_name="s",
                                  num_cores=2, num_subcores=16)
    scratch = [
        pltpu.VMEM_SHARED((NROWS, HID), jnp.float32),
        pltpu.VMEM((8, 2, CHUNK), jnp.int32),
        pltpu.VMEM((CHUNK, HID), jnp.float32),
        pltpu.VMEM((16, HID), jnp.float32),
        [pltpu.SemaphoreType.DMA] * 8,
        [pltpu.SemaphoreType.DMA] * 4,
    ]

    def body(sdr, parts, acc, sd_b, ones_v, zrow_v, isems, ssems):
        _sc_cnt_impl(sdr, parts, acc, sd_b, ones_v, zrow_v, isems, ssems)

    return pl.kernel(body,
                     out_type=[jax.ShapeDtypeStruct((2, NROWS, HID),
                                                    jnp.float32)],
                     mesh=mesh, scratch_types=scratch)


def _make_sc_agg(round0):
    mesh = plsc.VectorSubcoreMesh(core_axis_name="c", subcore_axis_name="s",
                                  num_cores=2, num_subcores=16)
    out_type = [jax.ShapeDtypeStruct((2, NROWS, HID), jnp.float32)]
    scratch = [
        pltpu.VMEM_SHARED((NROWS, HID), jnp.float32),
        pltpu.VMEM((8, 2, CHUNK), jnp.int32),
        pltpu.VMEM((4, CHUNK, HID), jnp.float32),
        pltpu.VMEM((16, HID), jnp.float32),
        [pltpu.SemaphoreType.DMA] * 8,
        [pltpu.SemaphoreType.DMA] * 4,
        [pltpu.SemaphoreType.DMA] * 4,
    ]
    if round0:
        scratch.append(pltpu.VMEM((4, CHUNK), jnp.int32))

        def body(h, sdr, parts, acc, sd_b, rows_v, zrow_v, isems,
                 gsems, ssems, msk_b):
            _sc_agg_impl(True, h, sdr, parts, acc, sd_b, msk_b, rows_v,
                         zrow_v, isems, gsems, ssems)
    else:
        def body(h, sdr, parts, acc, sd_b, rows_v, zrow_v, isems, gsems,
                 ssems):
            _sc_agg_impl(False, h, sdr, parts, acc, sd_b, None, rows_v,
                         zrow_v, isems, gsems, ssems)

    return pl.kernel(body, out_type=out_type, mesh=mesh,
                     scratch_types=scratch)


def _a_body(x_ref, m_ref, br_ref, wls_ref, bls_ref, wln_ref, bln_ref,
            sx_ref, xn_ref):
    h0 = jnp.dot(x_ref[...], m_ref[...],
                 preferred_element_type=jnp.float32) + br_ref[...]
    h0 = jnp.maximum(h0, 0.0)
    sx_ref[...] = jnp.dot(h0, wls_ref[...],
                          preferred_element_type=jnp.float32) + bls_ref[...]
    xn_ref[...] = jnp.dot(h0, wln_ref[...],
                          preferred_element_type=jnp.float32) + bln_ref[...]


def _b0_body(sx_ref, p0_ref, p1_ref, h_ref):
    h_ref[...] = sx_ref[...] + p0_ref[0] + p1_ref[0]


def _sage_mid_body(p0, p1, c0, c1, h_ref, wl_ref, bl_ref, wr_ref, g_ref,
                   b_ref, out_ref):
    sagg = p0[0] + p1[0]
    cnt = c0[0][:, 0:1] + c1[0][:, 0:1]
    mean = sagg / jnp.maximum(cnt, 1.0)
    t = (jnp.dot(mean, wl_ref[...], preferred_element_type=jnp.float32)
         + bl_ref[...]
         + jnp.dot(h_ref[...], wr_ref[...],
                   preferred_element_type=jnp.float32))
    hr = jnp.maximum(t, 0.0)
    mu = jnp.mean(hr, axis=-1, keepdims=True)
    var = jnp.mean((hr - mu) ** 2, axis=-1, keepdims=True)
    out_ref[...] = (hr - mu) / jnp.sqrt(var + 1e-5) * g_ref[...] + b_ref[...]


def _sage_last_body(p0, p1, c0, c1, h_ref, wl_ref, bl_ref, wr_ref, w1_ref,
                    b1_ref, w2_ref, b2_ref, emb_ref, lp_ref):
    sagg = p0[0] + p1[0]
    cnt = c0[0][:, 0:1] + c1[0][:, 0:1]
    mean = sagg / jnp.maximum(cnt, 1.0)
    t = (jnp.dot(mean, wl_ref[...], preferred_element_type=jnp.float32)
         + bl_ref[...]
         + jnp.dot(h_ref[...], wr_ref[...],
                   preferred_element_type=jnp.float32))
    emb_ref[...] = t
    hr = jnp.maximum(t, 0.0)
    z1 = jnp.dot(hr, w1_ref[...],
                 preferred_element_type=jnp.float32) + b1_ref[...]
    z = jnp.dot(z1, w2_ref[...],
                preferred_element_type=jnp.float32) + b2_ref[...]
    m = jnp.max(z, axis=-1, keepdims=True)
    lse = m + jnp.log(jnp.sum(jnp.exp(z - m), axis=-1, keepdims=True))
    lp_ref[...] = z - lse


def _rows_spec(width):
    return pl.BlockSpec((RB, width), lambda i: (i, 0))


def _full_spec(shape):
    nd = len(shape)
    return pl.BlockSpec(shape, lambda i: (0,) * nd)


def _part_spec(part, width):
    return pl.BlockSpec((1, RB, width), lambda i, _p=part: (_p, i, 0))


def _conv_as_matmul(Wcs):
    co, ci, di, dj, oi, oj = np.meshgrid(
        np.arange(4), np.arange(4), np.arange(3), np.arange(3),
        np.arange(8), np.arange(8), indexing="ij")
    rows = (ci * 100 + (oi + di) * 10 + (oj + dj)).ravel()
    cols = (co * 64 + oi * 8 + oj).ravel()
    vals = Wcs[co.ravel(), ci.ravel(), di.ravel(), dj.ravel()]
    return jnp.zeros((400, 256), jnp.float32).at[rows, cols].set(vals)


def kernel(x, edge_index, batch, params):
    f32 = jnp.float32
    x2d = x.reshape(N, 400)
    M = _conv_as_matmul(params["Wcs"])
    br = jnp.repeat(params["bcs"], 64).reshape(1, 256)

    npad = EPAD - E
    pad_src = (jnp.arange(npad, dtype=jnp.int32) * 7919) % N
    pad_dst = N + jnp.arange(npad, dtype=jnp.int32) % (NROWS - N)
    src_p = jnp.concatenate([edge_index[0], pad_src]).reshape(EPAD // CHUNK,
                                                              1, CHUNK)
    dst_p = jnp.concatenate([edge_index[1], pad_dst]).reshape(EPAD // CHUNK,
                                                              1, CHUNK)
    sd_p = jnp.concatenate([src_p, dst_p], axis=1)

    sx, xn = pl.pallas_call(
        _a_body,
        grid=(GRID,),
        in_specs=[
            _rows_spec(400),
            _full_spec((400, 256)), _full_spec((1, 256)),
            _full_spec((256, HID)), _full_spec((1, HID)),
            _full_spec((256, HID)), _full_spec((1, HID)),
        ],
        out_specs=[_rows_spec(HID), _rows_spec(HID)],
        out_shape=[jax.ShapeDtypeStruct((N, HID), f32)] * 2,
    )(x2d, M, br,
      params["W_lin_self"], params["b_lin_self"].reshape(1, HID),
      params["W_lin"], params["b_lin"].reshape(1, HID))

    sc_agg = _make_sc_agg(False)

    (parts0,) = _make_sc_agg(True)(xn, sd_p)
    (cnt,) = _make_sc_cnt()(sd_p)

    h = pl.pallas_call(
        _b0_body,
        grid=(GRID,),
        in_specs=[_rows_spec(HID), _part_spec(0, HID), _part_spec(1, HID)],
        out_specs=_rows_spec(HID),
        out_shape=jax.ShapeDtypeStruct((N, HID), f32),
    )(sx, parts0, parts0)

    hcur = h
    emb = lp = None
    for i in (1, 2, 3):
        (parts,) = sc_agg(hcur, sd_p)
        common_in = [_part_spec(0, HID), _part_spec(1, HID),
                     _part_spec(0, HID), _part_spec(1, HID), _rows_spec(HID),
                     _full_spec((HID, HID)), _full_spec((1, HID)),
                     _full_spec((HID, HID))]
        wl = params["Wl%d" % i]
        bl = params["bl%d" % i].reshape(1, HID)
        wr = params["Wr%d" % i]
        if i != 3:
            hcur = pl.pallas_call(
                _sage_mid_body,
                grid=(GRID,),
                in_specs=common_in + [_full_spec((1, HID)),
                                      _full_spec((1, HID))],
                out_specs=_rows_spec(HID),
                out_shape=jax.ShapeDtypeStruct((N, HID), f32),
            )(parts, parts, cnt, cnt, hcur, wl, bl, wr,
              params["ln%d_g" % i].reshape(1, HID),
              params["ln%d_b" % i].reshape(1, HID))
        else:
            emb, lp = pl.pallas_call(
                _sage_last_body,
                grid=(GRID,),
                in_specs=common_in + [
                    _full_spec((HID, HID)), _full_spec((1, HID)),
                    _full_spec((HID, 16)), _full_spec((1, 16))],
                out_specs=[_rows_spec(HID), _rows_spec(16)],
                out_shape=[jax.ShapeDtypeStruct((N, HID), f32),
                           jax.ShapeDtypeStruct((N, 16), f32)],
            )(parts, parts, cnt, cnt, hcur, wl, bl, wr,
              params["W1"], params["b1"].reshape(1, HID),
              params["W2"], params["b2"].reshape(1, 16))

    return emb, lp


# speedup vs baseline: 7.0918x; 1.0007x over previous
import numpy as np
import jax
import jax.numpy as jnp
from jax import lax
from jax.experimental import pallas as pl
from jax.experimental.pallas import tpu as pltpu
from jax.experimental.pallas import tpu_sc as plsc

N = 10000
E = 320000
HID = 128
NROWS = 10240
TROWS = NROWS // 16
CHUNK = 80
NCH = 128
NTILES = 32
EPAD = NTILES * NCH * CHUNK
RB = 400
GRID = N // RB


def _sc_agg_impl(round0, h, sdr, parts, acc, sd_b, msk_b, rows_v, zrow_v,
                 isems, gsems, ssems):
    c = lax.axis_index("c")
    s = lax.axis_index("s")
    base = (c * 16 + s) * NCH

    def zbody(i, carry):
        for l in range(8):
            zrow_v[i, pl.ds(l * 16, 16)] = jnp.zeros((16,), jnp.float32)
        return carry
    lax.fori_loop(0, 16, zbody, 0)

    rb = s * TROWS

    def istart(j, q):
        pltpu.async_copy(sdr.at[base + j], sd_b.at[q], isems[q])

    def iwait(q):
        pltpu.make_async_copy(sdr.at[0], sd_b.at[q], isems[q]).wait()

    def mcomp(q, mb):
        for l in range(CHUNK // 16):
            s16 = sd_b[q, 0, pl.ds(l * 16, 16)]
            d16 = sd_b[q, 1, pl.ds(l * 16, 16)]
            t16 = jnp.int32(N) + jnp.bitwise_and(d16, jnp.int32(127))
            msk_b[mb, pl.ds(l * 16, 16)] = jnp.where(s16 == d16, t16, d16)

    def gstart(j, b, q):
        pltpu.async_copy(h.at[sd_b.at[q, 0]], rows_v.at[b], gsems[b])

    def gwait(b):
        pltpu.make_async_copy(h.at[pl.ds(0, CHUNK)], rows_v.at[b],
                              gsems[b]).wait()

    def sstart(j, b, q):
        idx = msk_b.at[b] if round0 else sd_b.at[q, 1]
        pltpu.async_copy(rows_v.at[b], acc.at[idx], ssems[b], add=True)

    def swait(b):
        pltpu.make_async_copy(rows_v.at[b], acc.at[pl.ds(0, CHUNK)],
                              ssems[b]).wait()

    for q0 in range(6):
        istart(q0, q0)

    def ibody(k, carry):
        pltpu.sync_copy(zrow_v, acc.at[pl.ds(rb + k * 16, 16)])
        return carry
    lax.fori_loop(0, TROWS // 16, ibody, 0)

    for j0 in range(2):
        iwait(j0)
        if round0:
            mcomp(j0, j0)
        gstart(j0, j0, j0)
    plsc.subcore_barrier()

    def lbody(g, carry):
        for b8 in range(8):
            j = 8 * g + b8
            b = b8 % 4
            q = b8

            gwait(b)
            sstart(j, b, q)
            if b8 >= 2:
                swait((b8 - 2) % 4)
            else:
                pl.when(g >= 1)(lambda bb=(b8 - 2) % 4: swait(bb))

            def nxt(jq=(b8 + 2) % 8, jb=(b8 + 2) % 4, jj=j + 2):
                iwait(jq)
                if round0:
                    mcomp(jq, jb)
                gstart(jj, jb, jq)
            if b8 < 6:
                nxt()
            else:
                pl.when(g <= NCH // 8 - 2)(nxt)

            def pre(pq=(b8 + 6) % 8, pj=j + 6):
                istart(pj, pq)
            if b8 < 2:
                pre()
            else:
                pl.when(g <= NCH // 8 - 2)(pre)
        return carry
    lax.fori_loop(0, NCH // 8, lbody, 0)

    swait((NCH - 2) % 4)
    swait((NCH - 1) % 4)
    plsc.subcore_barrier()

    pltpu.sync_copy(acc.at[pl.ds(rb, TROWS)], parts.at[c, pl.ds(rb, TROWS)])


def _sc_cnt_impl(sdr, parts, acc, sd_b, ones_v, zrow_v, isems, ssems):
    c = lax.axis_index("c")
    s = lax.axis_index("s")
    base = (c * 16 + s) * NCH
    rb = s * TROWS

    for q0 in range(4):
        pltpu.async_copy(sdr.at[base + q0], sd_b.at[q0], isems[q0])

    def zbody(i, carry):
        for l in range(8):
            zrow_v[i, pl.ds(l * 16, 16)] = jnp.zeros((16,), jnp.float32)
        return carry
    lax.fori_loop(0, 16, zbody, 0)

    def obody(i, carry):
        for l in range(8):
            ones_v[i, pl.ds(l * 16, 16)] = jnp.ones((16,), jnp.float32)
        return carry
    lax.fori_loop(0, CHUNK, obody, 0)

    def ibody(k, carry):
        pltpu.sync_copy(zrow_v, acc.at[pl.ds(rb + k * 16, 16)])
        return carry
    lax.fori_loop(0, TROWS // 16, ibody, 0)

    plsc.subcore_barrier()

    def swait(s4):
        pltpu.make_async_copy(ones_v, acc.at[pl.ds(0, CHUNK)],
                              ssems[s4]).wait()

    def lbody(g, carry):
        for b8 in range(8):
            j = 8 * g + b8
            q = b8
            s4 = b8 % 4
            if b8 >= 4:
                swait(s4)
            else:
                pl.when(g >= 1)(lambda ss=s4: swait(ss))
            pltpu.make_async_copy(sdr.at[0], sd_b.at[q], isems[q]).wait()
            pltpu.async_copy(ones_v, acc.at[sd_b.at[q, 1]], ssems[s4],
                             add=True)

            def pre(pq=(b8 + 4) % 8, pj=j + 4):
                pltpu.async_copy(sdr.at[base + pj], sd_b.at[pq], isems[pq])
            if b8 < 4:
                pre()
            else:
                pl.when(g <= NCH // 8 - 2)(pre)
        return carry
    lax.fori_loop(0, NCH // 8, lbody, 0)

    for s4 in range(4):
        swait(s4)
    plsc.subcore_barrier()
    pltpu.sync_copy(acc.at[pl.ds(rb, TROWS)], parts.at[c, pl.ds(rb, TROWS)])


def _make_sc_cnt():
    mesh = plsc.VectorSubcoreMesh(core_axis_name="c", subcore_axis_name="s",
                                  num_cores=2, num_subcores=16)
    scratch = [
        pltpu.VMEM_SHARED((NROWS, HID), jnp.float32),
        pltpu.VMEM((8, 2, CHUNK), jnp.int32),
        pltpu.VMEM((CHUNK, HID), jnp.float32),
        pltpu.VMEM((16, HID), jnp.float32),
        [pltpu.SemaphoreType.DMA] * 8,
        [pltpu.SemaphoreType.DMA] * 4,
    ]

    def body(sdr, parts, acc, sd_b, ones_v, zrow_v, isems, ssems):
        _sc_cnt_impl(sdr, parts, acc, sd_b, ones_v, zrow_v, isems, ssems)

    return pl.kernel(body,
                     out_type=[jax.ShapeDtypeStruct((2, NROWS, HID),
                                                    jnp.float32)],
                     mesh=mesh, scratch_types=scratch)


def _make_sc_agg(round0):
    mesh = plsc.VectorSubcoreMesh(core_axis_name="c", subcore_axis_name="s",
                                  num_cores=2, num_subcores=16)
    out_type = [jax.ShapeDtypeStruct((2, NROWS, HID), jnp.float32)]
    scratch = [
        pltpu.VMEM_SHARED((NROWS, HID), jnp.float32),
        pltpu.VMEM((8, 2, CHUNK), jnp.int32),
        pltpu.VMEM((4, CHUNK, HID), jnp.float32),
        pltpu.VMEM((16, HID), jnp.float32),
        [pltpu.SemaphoreType.DMA] * 8,
        [pltpu.SemaphoreType.DMA] * 4,
        [pltpu.SemaphoreType.DMA] * 4,
    ]
    if round0:
        scratch.append(pltpu.VMEM((4, CHUNK), jnp.int32))

        def body(h, sdr, parts, acc, sd_b, rows_v, zrow_v, isems,
                 gsems, ssems, msk_b):
            _sc_agg_impl(True, h, sdr, parts, acc, sd_b, msk_b, rows_v,
                         zrow_v, isems, gsems, ssems)
    else:
        def body(h, sdr, parts, acc, sd_b, rows_v, zrow_v, isems, gsems,
                 ssems):
            _sc_agg_impl(False, h, sdr, parts, acc, sd_b, None, rows_v,
                         zrow_v, isems, gsems, ssems)

    return pl.kernel(body, out_type=out_type, mesh=mesh,
                     scratch_types=scratch)


def _a_body(x_ref, m_ref, br_ref, wls_ref, bls_ref, wln_ref, bln_ref,
            sx_ref, xn_ref):
    h0 = jnp.dot(x_ref[...], m_ref[...],
                 preferred_element_type=jnp.float32) + br_ref[...]
    h0 = jnp.maximum(h0, 0.0)
    sx_ref[...] = jnp.dot(h0, wls_ref[...],
                          preferred_element_type=jnp.float32) + bls_ref[...]
    xn_ref[...] = jnp.dot(h0, wln_ref[...],
                          preferred_element_type=jnp.float32) + bln_ref[...]


def _b0_body(sx_ref, p0_ref, p1_ref, h_ref):
    h_ref[...] = sx_ref[...] + p0_ref[0] + p1_ref[0]


def _sage_mid_body(p0, p1, c0, c1, h_ref, wl_ref, bl_ref, wr_ref, g_ref,
                   b_ref, out_ref):
    sagg = p0[0] + p1[0]
    cnt = c0[0][:, 0:1] + c1[0][:, 0:1]
    mean = sagg / jnp.maximum(cnt, 1.0)
    t = (jnp.dot(mean, wl_ref[...], preferred_element_type=jnp.float32)
         + bl_ref[...]
         + jnp.dot(h_ref[...], wr_ref[...],
                   preferred_element_type=jnp.float32))
    hr = jnp.maximum(t, 0.0)
    mu = jnp.mean(hr, axis=-1, keepdims=True)
    var = jnp.mean((hr - mu) ** 2, axis=-1, keepdims=True)
    out_ref[...] = (hr - mu) / jnp.sqrt(var + 1e-5) * g_ref[...] + b_ref[...]


def _sage_last_body(p0, p1, c0, c1, h_ref, wl_ref, bl_ref, wr_ref, w1_ref,
                    b1_ref, w2_ref, b2_ref, emb_ref, lp_ref):
    sagg = p0[0] + p1[0]
    cnt = c0[0][:, 0:1] + c1[0][:, 0:1]
    mean = sagg / jnp.maximum(cnt, 1.0)
    t = (jnp.dot(mean, wl_ref[...], preferred_element_type=jnp.float32)
         + bl_ref[...]
         + jnp.dot(h_ref[...], wr_ref[...],
                   preferred_element_type=jnp.float32))
    emb_ref[...] = t
    hr = jnp.maximum(t, 0.0)
    z1 = jnp.dot(hr, w1_ref[...],
                 preferred_element_type=jnp.float32) + b1_ref[...]
    z = jnp.dot(z1, w2_ref[...],
                preferred_element_type=jnp.float32) + b2_ref[...]
    m = jnp.max(z, axis=-1, keepdims=True)
    lse = m + jnp.log(jnp.sum(jnp.exp(z - m), axis=-1, keepdims=True))
    lp_ref[...] = z - lse


def _rows_spec(width):
    return pl.BlockSpec((RB, width), lambda i: (i, 0))


def _full_spec(shape):
    nd = len(shape)
    return pl.BlockSpec(shape, lambda i: (0,) * nd)


def _part_spec(part, width):
    return pl.BlockSpec((1, RB, width), lambda i, _p=part: (_p, i, 0))


def _conv_as_matmul(Wcs):
    co, ci, di, dj, oi, oj = np.meshgrid(
        np.arange(4), np.arange(4), np.arange(3), np.arange(3),
        np.arange(8), np.arange(8), indexing="ij")
    rows = (ci * 100 + (oi + di) * 10 + (oj + dj)).ravel()
    cols = (co * 64 + oi * 8 + oj).ravel()
    vals = Wcs[co.ravel(), ci.ravel(), di.ravel(), dj.ravel()]
    return jnp.zeros((400, 256), jnp.float32).at[rows, cols].set(vals)


def kernel(x, edge_index, batch, params):
    f32 = jnp.float32
    x2d = x.reshape(N, 400)
    M = _conv_as_matmul(params["Wcs"])
    br = jnp.repeat(params["bcs"], 64).reshape(1, 256)

    npad = EPAD - E
    pad_src = (jnp.arange(npad, dtype=jnp.int32) * 7919) % N
    pad_dst = N + jnp.arange(npad, dtype=jnp.int32) % (NROWS - N)
    src_p = jnp.concatenate([edge_index[0], pad_src]).reshape(EPAD // CHUNK,
                                                              1, CHUNK)
    dst_p = jnp.concatenate([edge_index[1], pad_dst]).reshape(EPAD // CHUNK,
                                                              1, CHUNK)
    sd_p = jnp.concatenate([src_p, dst_p], axis=1)

    sx, xn = pl.pallas_call(
        _a_body,
        grid=(GRID,),
        in_specs=[
            _rows_spec(400),
            _full_spec((400, 256)), _full_spec((1, 256)),
            _full_spec((256, HID)), _full_spec((1, HID)),
            _full_spec((256, HID)), _full_spec((1, HID)),
        ],
        out_specs=[_rows_spec(HID), _rows_spec(HID)],
        out_shape=[jax.ShapeDtypeStruct((N, HID), f32)] * 2,
    )(x2d, M, br,
      params["W_lin_self"], params["b_lin_self"].reshape(1, HID),
      params["W_lin"], params["b_lin"].reshape(1, HID))

    sc_agg = _make_sc_agg(False)

    (parts0,) = _make_sc_agg(True)(xn, sd_p)
    (cnt,) = _make_sc_cnt()(sd_p)

    h = pl.pallas_call(
        _b0_body,
        grid=(GRID,),
        in_specs=[_rows_spec(HID), _part_spec(0, HID), _part_spec(1, HID)],
        out_specs=_rows_spec(HID),
        out_shape=jax.ShapeDtypeStruct((N, HID), f32),
    )(sx, parts0, parts0)

    hcur = h
    emb = lp = None
    for i in (1, 2, 3):
        (parts,) = sc_agg(hcur, sd_p)
        common_in = [_part_spec(0, HID), _part_spec(1, HID),
                     _part_spec(0, HID), _part_spec(1, HID), _rows_spec(HID),
                     _full_spec((HID, HID)), _full_spec((1, HID)),
                     _full_spec((HID, HID))]
        wl = params["Wl%d" % i]
        bl = params["bl%d" % i].reshape(1, HID)
        wr = params["Wr%d" % i]
        if i != 3:
            hcur = pl.pallas_call(
                _sage_mid_body,
                grid=(GRID,),
                in_specs=common_in + [_full_spec((1, HID)),
                                      _full_spec((1, HID))],
                out_specs=_rows_spec(HID),
                out_shape=jax.ShapeDtypeStruct((N, HID), f32),
            )(parts, parts, cnt, cnt, hcur, wl, bl, wr,
              params["ln%d_g" % i].reshape(1, HID),
              params["ln%d_b" % i].reshape(1, HID))
        else:
            emb, lp = pl.pallas_call(
                _sage_last_body,
                grid=(GRID,),
                in_specs=common_in + [
                    _full_spec((HID, HID)), _full_spec((1, HID)),
                    _full_spec((HID, 16)), _full_spec((1, 16))],
                out_specs=[_rows_spec(HID), _rows_spec(16)],
                out_shape=[jax.ShapeDtypeStruct((N, HID), f32),
                           jax.ShapeDtypeStruct((N, 16), f32)],
            )(parts, parts, cnt, cnt, hcur, wl, bl, wr,
              params["W1"], params["b1"].reshape(1, HID),
              params["W2"], params["b2"].reshape(1, 16))

    return emb, lp
